# baseline (speedup 1.0000x reference)
import functools

import jax
import jax.numpy as jnp
from jax import lax
from jax.experimental import pallas as pl
from jax.experimental.pallas import tpu as pltpu
from jax.experimental.pallas import tpu_sc as plsc

BEAM = 4
VOCAB = 100000
PAD = 1
ROWS = 512
BSZ = ROWS // BEAM
K2 = 2 * BEAM

NC, NS, L = 2, 16, 16
NW = NC * NS
HA = 48000
HB = VOCAB - HA
UNITS = ROWS * 2
UNITS_PER_W = UNITS // NW
PAIRS = UNITS_PER_W // 2
GROUP = 5
UNROLL = 5

CAPJ = 32
NCAND = GROUP * CAPJ * L

NEG = float("-inf")


def _merge_slice(x, iv, V, I):
    xd, xid = plsc.sort_key_val(x, iv, descending=True)
    keep = xd > V
    V2 = jnp.where(keep, xd, V)
    I2 = jnp.where(keep, xid, I)
    Vn, In = plsc.sort_key_val(V2, I2, descending=False)
    return (Vn, In)


def _sc_body(logits_hbm, stat_hbm, idx_hbm,
             buf0, buf1, candi, vstage, istage, sem0, sem1):
    wid = lax.axis_index("s") * NC + lax.axis_index("c")
    lane = lax.iota(jnp.int32, L)
    ubase = wid * UNITS_PER_W

    def zi(j, _):
        candi[pl.ds(j * L, L)] = jnp.zeros((L,), jnp.int32)
        return 0

    lax.fori_loop(0, NCAND // L, zi, 0)

    def process(rowbuf, u, ulen, even):
        neg = jnp.full((L,), NEG, jnp.float32)

        def a_body(base, ms):
            ma, mb = ms
            xs = [rowbuf[pl.ds(base + j * L, L)] for j in range(GROUP)]
            ta = jnp.maximum(jnp.maximum(xs[0], xs[1]),
                             jnp.maximum(xs[2], xs[3]))
            return (jnp.maximum(ma, ta), jnp.maximum(mb, xs[4]))

        ma, mb = plsc.parallel_loop(
            0, ulen, step=GROUP * L, unroll=2 * UNROLL,
            carry=(neg, neg))(a_body)
        m = jnp.maximum(ma, mb)

        t = m
        for k in (8, 4, 2, 1):
            perm = (lane + k) & (L - 1)
            t = jnp.minimum(t, t.at[perm].get(mode="promise_in_bounds"))
        thr = t

        wrap = CAPJ * L - 1

        def b_body(base, carry):
            sacc = list(carry[:GROUP])
            cnts = list(carry[GROUP:])
            for j in range(GROUP):
                x = rowbuf[pl.ds(base + j * L, L)]
                sacc[j] = sacc[j] + jnp.exp(x - m)
                hitm = x >= thr
                idxv = base + j * L + lane
                pos = (cnts[j] & wrap) + j * (CAPJ * L)
                plsc.store_scatter(candi, [pos], idxv, mask=hitm)
                cnts[j] = cnts[j] + jnp.where(hitm, L, 0)
            return (*sacc, *cnts)

        z = jnp.zeros((L,), jnp.float32)
        out = plsc.parallel_loop(
            0, ulen, step=GROUP * L, unroll=2 * UNROLL,
            carry=(z,) * GROUP + (lane,) * GROUP)(b_body)
        sacc = out[:GROUP]
        cnts = out[GROUP:]
        s = (sacc[0] + sacc[1]) + (sacc[2] + sacc[3]) + sacc[4]

        cmax = cnts[0]
        for j in range(1, GROUP):
            cmax = jnp.maximum(cmax, cnts[j])
        overflow = jnp.any(cmax - lane > wrap)

        padloc = PAD if even else -1

        def fast(ops):
            V, I = ops
            for j in range(GROUP):
                nslots = jnp.max(cnts[j] - lane) // L
                rb = j * (CAPJ * L)

                def mbody(q, VI, j=j, rb=rb):
                    V, I = VI
                    iv = candi[pl.ds(rb + q * L, L)]
                    valid = (cnts[j] - lane) > q * L
                    xv = plsc.load_gather(rowbuf, [iv])
                    x = jnp.where(valid & (iv != padloc), xv, NEG)
                    return _merge_slice(x, iv, V, I)

                V, I = lax.fori_loop(0, nslots, mbody, (V, I))
            return (V, I)

        def slow(ops):
            def mslice(ti, VI):
                V, I = VI
                x = rowbuf[pl.ds(ti * L, L)]
                iv = ti * L + lane
                x = jnp.where(iv == padloc, NEG, x)
                return _merge_slice(x, iv, V, I)

            return lax.fori_loop(0, ulen // L, mslice, ops)

        vinit = (jnp.full((L,), NEG, jnp.float32), jnp.zeros((L,), jnp.int32))
        V, I = lax.cond(overflow, slow, fast, vinit)

        vstage[0] = m
        vstage[1] = s
        vstage[2] = V
        istage[...] = I if even else I + HA
        pltpu.sync_copy(vstage, stat_hbm.at[u])
        pltpu.sync_copy(istage, idx_hbm.at[u])

    def src_even(u):
        return logits_hbm.at[u // 2, pl.ds(0, HA)]

    def src_odd(u):
        return logits_hbm.at[u // 2, pl.ds(HA, HB)]

    pltpu.make_async_copy(src_even(ubase), buf0, sem0).start()

    def pair_body(p, _):
        ue = ubase + 2 * p
        pltpu.make_async_copy(src_odd(ue + 1), buf1, sem1).start()
        pltpu.make_async_copy(src_even(ue), buf0, sem0).wait()
        process(buf0, ue, HA, True)

        @pl.when(p < PAIRS - 1)
        def _prefetch():
            pltpu.make_async_copy(src_even(ue + 2), buf0, sem0).start()

        pltpu.make_async_copy(src_odd(ue + 1), buf1, sem1).wait()
        process(buf1, ue + 1, HB, False)
        return 0

    lax.fori_loop(0, PAIRS, pair_body, 0)


_sc_topk = functools.partial(
    pl.kernel,
    out_type=[
        jax.ShapeDtypeStruct((UNITS, 3, L), jnp.float32),
        jax.ShapeDtypeStruct((UNITS, L), jnp.int32),
    ],
    mesh=plsc.VectorSubcoreMesh(
        core_axis_name="c", subcore_axis_name="s",
        num_cores=NC, num_subcores=NS),
    scratch_types=[
        pltpu.VMEM((HA,), jnp.float32),
        pltpu.VMEM((HB,), jnp.float32),
        pltpu.VMEM((NCAND,), jnp.int32),
        pltpu.VMEM((3, L), jnp.float32),
        pltpu.VMEM((L,), jnp.int32),
        pltpu.SemaphoreType.DMA,
        pltpu.SemaphoreType.DMA,
    ],
    compiler_params=pltpu.CompilerParams(needs_layout_passes=False),
)(_sc_body)


NLANE = 2 * BEAM * L


def _tc_merge_body(m_ref, s_ref, v_ref, i_ref, sc_ref, os_ref, ot_ref, ob_ref):
    m = m_ref[...]
    s = s_ref[...]
    V = v_ref[...]
    I = i_ref[...]
    sc = sc_ref[...]

    pos = lax.broadcasted_iota(jnp.int32, (BSZ, NLANE), 1)
    seg = pos // (2 * L)
    offs = jnp.zeros((BSZ, NLANE), jnp.float32)
    for j in range(BEAM):
        maskj = seg == j
        mj = jnp.max(jnp.where(maskj, m, NEG), axis=1, keepdims=True)
        sj = jnp.sum(jnp.where(maskj, s * jnp.exp(m - mj), 0.0),
                     axis=1, keepdims=True)
        offj = sc[:, j:j + 1] - (mj + jnp.log(sj))
        offs = jnp.where(maskj, offj, offs)

    cand = V + offs
    flat = I + seg * VOCAB
    big = jnp.int32(2**31 - 1)
    vals, flats = [], []
    for _ in range(K2):
        cur = jnp.max(cand, axis=1, keepdims=True)
        cf = jnp.min(jnp.where(cand == cur, flat, big), axis=1, keepdims=True)
        vals.append(cur)
        flats.append(cf)
        cand = jnp.where(flat == cf, NEG, cand)
    ts = jnp.concatenate(vals, axis=1)
    tf = jnp.concatenate(flats, axis=1)
    os_ref[...] = ts
    ot_ref[...] = tf % VOCAB
    ob_ref[...] = tf // VOCAB


_tc_merge = pl.pallas_call(
    _tc_merge_body,
    out_shape=(
        jax.ShapeDtypeStruct((BSZ, K2), jnp.float32),
        jax.ShapeDtypeStruct((BSZ, K2), jnp.int32),
        jax.ShapeDtypeStruct((BSZ, K2), jnp.int32),
    ),
)


def kernel(logits, scores):
    stat, idx = _sc_topk(logits)
    m4 = stat[:, 0, :].reshape(BSZ, NLANE)
    s4 = stat[:, 1, :].reshape(BSZ, NLANE)
    v4 = stat[:, 2, :].reshape(BSZ, NLANE)
    i4 = idx.reshape(BSZ, NLANE)
    sc4 = scores.reshape(BSZ, BEAM)
    return _tc_merge(m4, s4, v4, i4, sc4)


# speedup vs baseline: 1.1621x; 1.1621x over previous
import functools

import jax
import jax.numpy as jnp
from jax import lax
from jax.experimental import pallas as pl
from jax.experimental.pallas import tpu as pltpu
from jax.experimental.pallas import tpu_sc as plsc

BEAM = 4
VOCAB = 100000
PAD = 1
ROWS = 512
BSZ = ROWS // BEAM
K2 = 2 * BEAM

NC, NS, L = 2, 16, 16
NW = NC * NS
HA = 48000
HB = VOCAB - HA
UNITS = ROWS * 2
UNITS_PER_W = UNITS // NW
PAIRS = UNITS_PER_W // 2
GROUP = 5
UNROLL = 5

CAPJ = 32
NCAND = GROUP * CAPJ * L

NEG = float("-inf")


def _merge_slice(x, iv, V, I):
    xd, xid = plsc.sort_key_val(x, iv, descending=True)
    keep = xd > V
    V2 = jnp.where(keep, xd, V)
    I2 = jnp.where(keep, xid, I)
    Vn, In = plsc.sort_key_val(V2, I2, descending=False)
    return (Vn, In)


def _sc_body(logits_hbm, stat_hbm, idx_hbm,
             buf0, buf1, candi, vstage, istage, sem0, sem1):
    wid = lax.axis_index("s") * NC + lax.axis_index("c")
    lane = lax.iota(jnp.int32, L)
    ubase = wid * UNITS_PER_W

    def zi(j, _):
        candi[pl.ds(j * L, L)] = jnp.zeros((L,), jnp.int32)
        return 0

    lax.fori_loop(0, NCAND // L, zi, 0)

    def process(rowbuf, u, ulen, even):
        neg = jnp.full((L,), NEG, jnp.float32)

        def a_body(base, ms):
            ma, mb = ms
            xs = [rowbuf[pl.ds(base + j * L, L)] for j in range(GROUP)]
            ta = jnp.maximum(jnp.maximum(xs[0], xs[1]),
                             jnp.maximum(xs[2], xs[3]))
            return (jnp.maximum(ma, ta), jnp.maximum(mb, xs[4]))

        ma, mb = plsc.parallel_loop(
            0, ulen // 2, step=GROUP * L, unroll=UNROLL,
            carry=(neg, neg))(a_body)
        mh = jnp.maximum(ma, mb)

        t = mh
        for k in (8, 4, 2, 1):
            perm = (lane + k) & (L - 1)
            t = jnp.minimum(t, t.at[perm].get(mode="promise_in_bounds"))
        thr = t

        wrap = CAPJ * L - 1

        def b_body(base, carry):
            sacc = list(carry[:GROUP])
            cnts = list(carry[GROUP:])
            for j in range(GROUP):
                x = rowbuf[pl.ds(base + j * L, L)]
                sacc[j] = sacc[j] + jnp.exp(x - mh)
                hitm = x >= thr
                idxv = base + j * L + lane
                pos = (cnts[j] & wrap) + j * (CAPJ * L)
                plsc.store_scatter(candi, [pos], idxv, mask=hitm)
                cnts[j] = cnts[j] + jnp.where(hitm, L, 0)
            return (*sacc, *cnts)

        z = jnp.zeros((L,), jnp.float32)
        out = plsc.parallel_loop(
            0, ulen, step=GROUP * L, unroll=UNROLL,
            carry=(z,) * GROUP + (lane,) * GROUP)(b_body)
        sacc = out[:GROUP]
        cnts = out[GROUP:]
        s = (sacc[0] + sacc[1]) + (sacc[2] + sacc[3]) + sacc[4]

        cmax = cnts[0]
        for j in range(1, GROUP):
            cmax = jnp.maximum(cmax, cnts[j])
        overflow = jnp.any(cmax - lane > wrap)

        padloc = PAD if even else -1

        def fast(ops):
            V, I = ops
            for j in range(GROUP):
                nslots = jnp.max(cnts[j] - lane) // L
                rb = j * (CAPJ * L)

                def mbody(q, VI, j=j, rb=rb):
                    V, I = VI
                    iv = candi[pl.ds(rb + q * L, L)]
                    valid = (cnts[j] - lane) > q * L
                    xv = plsc.load_gather(rowbuf, [iv])
                    x = jnp.where(valid & (iv != padloc), xv, NEG)
                    return _merge_slice(x, iv, V, I)

                V, I = lax.fori_loop(0, nslots, mbody, (V, I))
            return (V, I)

        def slow(ops):
            def mslice(ti, VI):
                V, I = VI
                x = rowbuf[pl.ds(ti * L, L)]
                iv = ti * L + lane
                x = jnp.where(iv == padloc, NEG, x)
                return _merge_slice(x, iv, V, I)

            return lax.fori_loop(0, ulen // L, mslice, ops)

        vinit = (jnp.full((L,), NEG, jnp.float32), jnp.zeros((L,), jnp.int32))
        V, I = lax.cond(overflow, slow, fast, vinit)

        vstage[0] = mh
        vstage[1] = s
        vstage[2] = V
        istage[...] = I if even else I + HA
        pltpu.sync_copy(vstage, stat_hbm.at[u])
        pltpu.sync_copy(istage, idx_hbm.at[u])

    def src_even(u):
        return logits_hbm.at[u // 2, pl.ds(0, HA)]

    def src_odd(u):
        return logits_hbm.at[u // 2, pl.ds(HA, HB)]

    pltpu.make_async_copy(src_even(ubase), buf0, sem0).start()

    def pair_body(p, _):
        ue = ubase + 2 * p
        pltpu.make_async_copy(src_odd(ue + 1), buf1, sem1).start()
        pltpu.make_async_copy(src_even(ue), buf0, sem0).wait()
        process(buf0, ue, HA, True)

        @pl.when(p < PAIRS - 1)
        def _prefetch():
            pltpu.make_async_copy(src_even(ue + 2), buf0, sem0).start()

        pltpu.make_async_copy(src_odd(ue + 1), buf1, sem1).wait()
        process(buf1, ue + 1, HB, False)
        return 0

    lax.fori_loop(0, PAIRS, pair_body, 0)


_sc_topk = functools.partial(
    pl.kernel,
    out_type=[
        jax.ShapeDtypeStruct((UNITS, 3, L), jnp.float32),
        jax.ShapeDtypeStruct((UNITS, L), jnp.int32),
    ],
    mesh=plsc.VectorSubcoreMesh(
        core_axis_name="c", subcore_axis_name="s",
        num_cores=NC, num_subcores=NS),
    scratch_types=[
        pltpu.VMEM((HA,), jnp.float32),
        pltpu.VMEM((HB,), jnp.float32),
        pltpu.VMEM((NCAND,), jnp.int32),
        pltpu.VMEM((3, L), jnp.float32),
        pltpu.VMEM((L,), jnp.int32),
        pltpu.SemaphoreType.DMA,
        pltpu.SemaphoreType.DMA,
    ],
    compiler_params=pltpu.CompilerParams(needs_layout_passes=False),
)(_sc_body)


NLANE = 2 * BEAM * L


def _tc_merge_body(m_ref, s_ref, v_ref, i_ref, sc_ref, os_ref, ot_ref, ob_ref):
    m = m_ref[...]
    s = s_ref[...]
    V = v_ref[...]
    I = i_ref[...]
    sc = sc_ref[...]

    pos = lax.broadcasted_iota(jnp.int32, (BSZ, NLANE), 1)
    seg = pos // (2 * L)
    offs = jnp.zeros((BSZ, NLANE), jnp.float32)
    for j in range(BEAM):
        maskj = seg == j
        mj = jnp.max(jnp.where(maskj, m, NEG), axis=1, keepdims=True)
        sj = jnp.sum(jnp.where(maskj, s * jnp.exp(m - mj), 0.0),
                     axis=1, keepdims=True)
        offj = sc[:, j:j + 1] - (mj + jnp.log(sj))
        offs = jnp.where(maskj, offj, offs)

    cand = V + offs
    flat = I + seg * VOCAB
    big = jnp.int32(2**31 - 1)
    vals, flats = [], []
    for _ in range(K2):
        cur = jnp.max(cand, axis=1, keepdims=True)
        cf = jnp.min(jnp.where(cand == cur, flat, big), axis=1, keepdims=True)
        vals.append(cur)
        flats.append(cf)
        cand = jnp.where(flat == cf, NEG, cand)
    ts = jnp.concatenate(vals, axis=1)
    tf = jnp.concatenate(flats, axis=1)
    os_ref[...] = ts
    ot_ref[...] = tf % VOCAB
    ob_ref[...] = tf // VOCAB


_tc_merge = pl.pallas_call(
    _tc_merge_body,
    out_shape=(
        jax.ShapeDtypeStruct((BSZ, K2), jnp.float32),
        jax.ShapeDtypeStruct((BSZ, K2), jnp.int32),
        jax.ShapeDtypeStruct((BSZ, K2), jnp.int32),
    ),
)


def kernel(logits, scores):
    stat, idx = _sc_topk(logits)
    m4 = stat[:, 0, :].reshape(BSZ, NLANE)
    s4 = stat[:, 1, :].reshape(BSZ, NLANE)
    v4 = stat[:, 2, :].reshape(BSZ, NLANE)
    i4 = idx.reshape(BSZ, NLANE)
    sc4 = scores.reshape(BSZ, BEAM)
    return _tc_merge(m4, s4, v4, i4, sc4)
